# baseline (speedup 1.0000x reference)
import functools
import math

import jax
import jax.numpy as jnp
from jax import lax
from jax.experimental import pallas as pl

EMB = 32
ATT = 16
DEG = 16
NREL = 100
RPAD = 128

NODES = 40
SLOTS = NODES * (DEG + 1)

_F32 = jnp.float32
_BF16 = jnp.bfloat16


def _hl(x):
    hi = x.astype(_BF16)
    lo = (x - hi.astype(_F32)).astype(_BF16)
    return hi, lo


def _dot_e(x, exact_bf16):
    xh, xl = _hl(x)
    return (jnp.dot(xh, exact_bf16, preferred_element_type=_F32)
            + jnp.dot(xl, exact_bf16, preferred_element_type=_F32))


def _dg3(a, b, dims):
    ah, al = _hl(a)
    bh, bl = _hl(b)
    dg = functools.partial(lax.dot_general, dimension_numbers=dims,
                           preferred_element_type=_F32)
    return dg(ah, bh) + dg(ah, bl) + dg(al, bh)

_MM = (((1,), (0,)), ((), ()))
_MMT = (((1,), (1,)), ((), ()))


def _attn_block(h_ref, msg_ref, rl_ref, mrl_ref, mt_ref, wself_ref,
                qh_ref, ql_ref, kh_ref, kl_ref, vh_ref, vl_ref,
                ffnwt_ref, ffnb_ref, out_ref):
    curr = _dg3(h_ref[...], wself_ref[...], _MM)
    e_all = jnp.concatenate([curr, msg_ref[...]], axis=0)

    rl = rl_ref[0]
    mrl = mrl_ref[0]
    mt = mt_ref[0]
    idx_qv = jnp.concatenate([rl, mrl], axis=0)
    idx_k = jnp.concatenate([jnp.full((NODES, 1), NREL, rl.dtype), mt], axis=0)

    lane_r = lax.broadcasted_iota(jnp.int32, (SLOTS, RPAD), 1)
    o_qv = (idx_qv == lane_r).astype(_BF16)
    o_k = (idx_k == lane_r).astype(_BF16)

    r_sub = lax.broadcasted_iota(jnp.int32, (EMB, EMB * ATT), 0)
    r_lane = lax.broadcasted_iota(jnp.int32, (EMB, EMB * ATT), 1)
    rep_mat = ((r_lane // ATT) == r_sub).astype(_BF16)
    f_sub = lax.broadcasted_iota(jnp.int32, (EMB * ATT, ATT), 0)
    f_lane = lax.broadcasted_iota(jnp.int32, (EMB * ATT, ATT), 1)
    fold_mat = ((f_sub % ATT) == f_lane).astype(_BF16)

    e_rep = _dot_e(e_all, rep_mat)

    def project(onehot, th_ref, tl_ref):
        wg = (jnp.dot(onehot, th_ref[...], preferred_element_type=_F32)
              + jnp.dot(onehot, tl_ref[...], preferred_element_type=_F32))
        return _dot_e(wg * e_rep, fold_mat)

    q_all = project(o_qv, qh_ref, ql_ref)
    k_all = project(o_k, kh_ref, kl_ref)
    v_all = project(o_qv, vh_ref, vl_ref)

    s_full = _dg3(q_all, k_all, _MMT) * (1.0 / math.sqrt(ATT))

    i_sub = lax.broadcasted_iota(jnp.int32, (SLOTS, 1), 0)
    i_lane = lax.broadcasted_iota(jnp.int32, (1, SLOTS), 1)
    node_s = jnp.where(i_sub < NODES, i_sub, (i_sub - NODES) >> 4)
    node_t = jnp.where(i_lane < NODES, i_lane, (i_lane - NODES) >> 4)
    mask = node_s == node_t

    neg = jnp.where(mask, s_full, -1e30)
    m = jnp.max(neg, axis=0, keepdims=True)
    ex = jnp.exp(neg - m)
    denom = jnp.sum(ex, axis=0, keepdims=True)
    is_self = (i_sub < NODES).astype(_F32)
    numer = jnp.sum(ex * is_self, axis=0, keepdims=True)
    attn0 = numer / denom

    n_sub = lax.broadcasted_iota(jnp.int32, (NODES, 1), 0)
    sel = (node_t == n_sub).astype(_F32) * attn0
    pooled = _dg3(sel, v_all, _MM)

    out_ref[...] = _dg3(pooled, ffnwt_ref[...], _MM) + ffnb_ref[...]


def kernel(h, msg, r_label, msg_type, msg_r_label, self_loop_weight,
           relational_Q, relational_K, relational_V, ffn_w, ffn_b):
    bnum = h.shape[0]
    nblk = bnum // NODES
    inp = h.shape[1]

    msg2d = msg.reshape(bnum * DEG, EMB)
    rl3 = r_label.astype(jnp.int32).reshape(nblk, NODES, 1)
    mrl3 = msg_r_label.astype(jnp.int32).reshape(nblk, NODES * DEG, 1)
    mt3 = msg_type.astype(jnp.int32).reshape(nblk, NODES * DEG, 1)

    def padtbl(t):
        flat = t.reshape(NREL + 1, EMB * ATT)
        flat = jnp.concatenate(
            [flat, jnp.zeros((RPAD - (NREL + 1), EMB * ATT), flat.dtype)], axis=0)
        hi = flat.astype(_BF16)
        lo = (flat - hi.astype(_F32)).astype(_BF16)
        return hi, lo

    qth, qtl = padtbl(relational_Q)
    kth, ktl = padtbl(relational_K)
    vth, vtl = padtbl(relational_V)
    ffn_wt = ffn_w.T
    ffn_b2 = ffn_b.reshape(1, EMB)

    full = lambda shape: pl.BlockSpec(shape, lambda i: (0,) * len(shape))
    out = pl.pallas_call(
        _attn_block,
        grid=(nblk,),
        in_specs=[
            pl.BlockSpec((NODES, inp), lambda i: (i, 0)),
            pl.BlockSpec((NODES * DEG, EMB), lambda i: (i, 0)),
            pl.BlockSpec((1, NODES, 1), lambda i: (i, 0, 0)),
            pl.BlockSpec((1, NODES * DEG, 1), lambda i: (i, 0, 0)),
            pl.BlockSpec((1, NODES * DEG, 1), lambda i: (i, 0, 0)),
            full((inp, EMB)),
            full((RPAD, EMB * ATT)),
            full((RPAD, EMB * ATT)),
            full((RPAD, EMB * ATT)),
            full((RPAD, EMB * ATT)),
            full((RPAD, EMB * ATT)),
            full((RPAD, EMB * ATT)),
            full((ATT, EMB)),
            full((1, EMB)),
        ],
        out_specs=pl.BlockSpec((NODES, EMB), lambda i: (i, 0)),
        out_shape=jax.ShapeDtypeStruct((bnum, EMB), jnp.float32),
    )(h, msg2d, rl3, mrl3, mt3, self_loop_weight,
      qth, qtl, kth, ktl, vth, vtl, ffn_wt, ffn_b2)
    return out
